# baseline (speedup 1.0000x reference)
import functools

import jax
import jax.numpy as jnp
import numpy as np
from jax.experimental import pallas as pl
from jax.experimental.pallas import tpu as pltpu

_N = 10000
_T = 21
_F_IN = 128
_HID = 32
_BN = 384
_SCALE = 1.0 / (1.0 + 1e-5) ** 0.5


def _glu_t(Y):
    P = Y[0:32, :]
    Q = Y[32:64, :]
    R = Y[64:96, :]
    return jax.nn.relu(P * jax.nn.sigmoid(Q) + R)


def _tap_stack(H, t_out):
    L = t_out * _BN
    return jnp.concatenate(
        [H[:, 0:L], H[:, _BN:_BN + L], H[:, 2 * _BN:2 * _BN + L]], axis=0)


def _stgcn_block(x_ref, mask_ref, w1_ref, w234_ref, wab_ref, b14_ref,
                 bab_ref, lw_ref, lb_ref, out_ref, acc_ref, *, nblocks):
    i = pl.program_id(0)

    xb = x_ref[...].astype(jnp.bfloat16)
    X3 = jnp.transpose(xb, (0, 2, 1))
    xT = jnp.concatenate([X3[t] for t in range(_T)], axis=1)

    dot = functools.partial(jnp.dot, preferred_element_type=jnp.float32)
    A0 = dot(w1_ref[0], xT)
    A1 = dot(w1_ref[1], xT)
    A2 = dot(w1_ref[2], xT)
    L1 = 19 * _BN
    Y1 = (A0[:, 0:L1] + A1[:, _BN:_BN + L1] + A2[:, 2 * _BN:2 * _BN + L1]
          + b14_ref[0])
    H1 = _glu_t(Y1)
    Tc = jax.nn.relu(dot(wab_ref[0], H1) + bab_ref[0])
    H2 = _glu_t(dot(w234_ref[0], _tap_stack(Tc, 17)) + b14_ref[1]) * _SCALE
    H3 = _glu_t(dot(w234_ref[1], _tap_stack(H2, 15)) + b14_ref[2])
    Tc2 = jax.nn.relu(dot(wab_ref[1], H3) + bab_ref[1])
    H4 = _glu_t(dot(w234_ref[2], _tap_stack(Tc2, 13)) + b14_ref[3])

    mask = jnp.concatenate([mask_ref[0]] * 13, axis=1)
    H4 = jnp.where(mask > 0, H4, 0.0)
    part = jnp.sum(H4, axis=0, keepdims=True)

    @pl.when(i == 0)
    def _init():
        acc_ref[...] = jnp.zeros_like(acc_ref)

    acc_ref[...] += part

    @pl.when(i == nblocks - 1)
    def _finish():
        acc = acc_ref[...]
        a13 = jnp.concatenate(
            [acc[:, t * _BN:(t + 1) * _BN] for t in range(13)], axis=0)
        s = jnp.sum(a13, axis=1, keepdims=True)
        out = jnp.sum(s * lw_ref[...], axis=0, keepdims=True)
        out_ref[...] = out * (_SCALE / (_N * _HID)) + lb_ref[...]


def kernel(x, edge_index, edge_weight, tc1a, cheb_a, tc2a, tc1b, cheb_b, tc2b,
           lin_w, lin_b):
    del edge_index, edge_weight
    cat1 = jnp.concatenate(
        [tc1a[0].reshape(_HID, -1), tc1a[2].reshape(_HID, -1),
         tc1a[4].reshape(_HID, -1)], axis=0)
    W1 = jnp.transpose(cat1.reshape(96, _F_IN, 3), (2, 0, 1)).astype(jnp.bfloat16)
    cat234 = jnp.concatenate(
        [w.reshape(_HID, -1) for w in
         (tc2a[0], tc2a[2], tc2a[4], tc1b[0], tc1b[2], tc1b[4],
          tc2b[0], tc2b[2], tc2b[4])], axis=0)
    W234 = jnp.transpose(cat234.reshape(288, _HID, 3), (0, 2, 1)).reshape(3, 96, 96)
    B14 = jnp.concatenate(
        [tc1a[1], tc1a[3], tc1a[5], tc2a[1], tc2a[3], tc2a[5],
         tc1b[1], tc1b[3], tc1b[5], tc2b[1], tc2b[3], tc2b[5]]).reshape(4, 96, 1)
    Wab = jnp.transpose(jnp.stack([cheb_a[0], cheb_b[0]]), (0, 2, 1))
    Bab = jnp.concatenate([cheb_a[1], cheb_b[1]]).reshape(2, _HID, 1)
    lb = lin_b.reshape(1, -1)

    nblocks = -(-_N // _BN)
    mask = np.arange(nblocks * _BN) < _N
    mask = jnp.asarray(mask.astype(np.float32).reshape(nblocks, 1, _BN))

    full = lambda a: pl.BlockSpec(a.shape, lambda *_: tuple(0 for _ in a.shape))
    out = pl.pallas_call(
        functools.partial(_stgcn_block, nblocks=nblocks),
        grid=(nblocks,),
        in_specs=[
            pl.BlockSpec((_T, _BN, _F_IN), lambda i: (0, i, 0)),
            pl.BlockSpec((1, 1, _BN), lambda i: (i, 0, 0)),
            full(W1), full(W234), full(Wab), full(B14), full(Bab),
            full(lin_w), full(lb),
        ],
        out_specs=pl.BlockSpec((1, lin_w.shape[1]), lambda i: (0, 0)),
        out_shape=jax.ShapeDtypeStruct((1, lin_w.shape[1]), jnp.float32),
        scratch_shapes=[pltpu.VMEM((1, 13 * _BN), jnp.float32)],
    )(x, mask, W1, W234, Wab, B14, Bab, lin_w, lb)
    return out[0]


# speedup vs baseline: 1.0684x; 1.0684x over previous
import functools

import jax
import jax.numpy as jnp
import numpy as np
from jax.experimental import pallas as pl
from jax.experimental.pallas import tpu as pltpu

_N = 10000
_T = 21
_F_IN = 128
_HID = 32
_BN = 640
_SCALE = 1.0 / (1.0 + 1e-5) ** 0.5


def _glu_t(Y):
    P = Y[0:32, :]
    Q = Y[32:64, :]
    R = Y[64:96, :]
    return jax.nn.relu(P * jax.nn.sigmoid(Q) + R)


def _tap_stack(H, t_out):
    L = t_out * _BN
    return jnp.concatenate(
        [H[:, 0:L], H[:, _BN:_BN + L], H[:, 2 * _BN:2 * _BN + L]], axis=0)


def _stgcn_block(x_ref, mask_ref, w1_ref, w234_ref, wab_ref, b14_ref,
                 bab_ref, lw_ref, lb_ref, out_ref, acc_ref, *, nblocks):
    i = pl.program_id(0)

    xb = x_ref[...].astype(jnp.bfloat16)
    X3 = jnp.transpose(xb, (0, 2, 1))
    xT = jnp.concatenate([X3[t] for t in range(_T)], axis=1)

    dot = functools.partial(jnp.dot, preferred_element_type=jnp.float32)
    A0 = dot(w1_ref[0], xT)
    A1 = dot(w1_ref[1], xT)
    A2 = dot(w1_ref[2], xT)
    L1 = 19 * _BN
    Y1 = (A0[:, 0:L1] + A1[:, _BN:_BN + L1] + A2[:, 2 * _BN:2 * _BN + L1]
          + b14_ref[0])
    H1 = _glu_t(Y1)
    Tc = jax.nn.relu(dot(wab_ref[0], H1) + bab_ref[0])
    H2 = _glu_t(dot(w234_ref[0], _tap_stack(Tc, 17)) + b14_ref[1]) * _SCALE
    H3 = _glu_t(dot(w234_ref[1], _tap_stack(H2, 15)) + b14_ref[2])
    Tc2 = jax.nn.relu(dot(wab_ref[1], H3) + bab_ref[1])
    H4 = _glu_t(dot(w234_ref[2], _tap_stack(Tc2, 13)) + b14_ref[3])

    mask = jnp.concatenate([mask_ref[0]] * 13, axis=1)
    H4 = jnp.where(mask > 0, H4, 0.0)
    part = jnp.sum(H4, axis=0, keepdims=True)

    @pl.when(i == 0)
    def _init():
        acc_ref[...] = jnp.zeros_like(acc_ref)

    acc_ref[...] += part

    @pl.when(i == nblocks - 1)
    def _finish():
        acc = acc_ref[...]
        a13 = jnp.concatenate(
            [acc[:, t * _BN:(t + 1) * _BN] for t in range(13)], axis=0)
        s = jnp.sum(a13, axis=1, keepdims=True)
        out = jnp.sum(s * lw_ref[...], axis=0, keepdims=True)
        out_ref[...] = out * (_SCALE / (_N * _HID)) + lb_ref[...]


def kernel(x, edge_index, edge_weight, tc1a, cheb_a, tc2a, tc1b, cheb_b, tc2b,
           lin_w, lin_b):
    del edge_index, edge_weight
    cat1 = jnp.concatenate(
        [tc1a[0].reshape(_HID, -1), tc1a[2].reshape(_HID, -1),
         tc1a[4].reshape(_HID, -1)], axis=0)
    W1 = jnp.transpose(cat1.reshape(96, _F_IN, 3), (2, 0, 1)).astype(jnp.bfloat16)
    cat234 = jnp.concatenate(
        [w.reshape(_HID, -1) for w in
         (tc2a[0], tc2a[2], tc2a[4], tc1b[0], tc1b[2], tc1b[4],
          tc2b[0], tc2b[2], tc2b[4])], axis=0)
    W234 = jnp.transpose(cat234.reshape(288, _HID, 3), (0, 2, 1)).reshape(3, 96, 96)
    B14 = jnp.concatenate(
        [tc1a[1], tc1a[3], tc1a[5], tc2a[1], tc2a[3], tc2a[5],
         tc1b[1], tc1b[3], tc1b[5], tc2b[1], tc2b[3], tc2b[5]]).reshape(4, 96, 1)
    Wab = jnp.transpose(jnp.stack([cheb_a[0], cheb_b[0]]), (0, 2, 1))
    Bab = jnp.concatenate([cheb_a[1], cheb_b[1]]).reshape(2, _HID, 1)
    lb = lin_b.reshape(1, -1)

    nblocks = -(-_N // _BN)
    mask = np.arange(nblocks * _BN) < _N
    mask = jnp.asarray(mask.astype(np.float32).reshape(nblocks, 1, _BN))

    full = lambda a: pl.BlockSpec(a.shape, lambda *_: tuple(0 for _ in a.shape))
    out = pl.pallas_call(
        functools.partial(_stgcn_block, nblocks=nblocks),
        grid=(nblocks,),
        in_specs=[
            pl.BlockSpec((_T, _BN, _F_IN), lambda i: (0, i, 0)),
            pl.BlockSpec((1, 1, _BN), lambda i: (i, 0, 0)),
            full(W1), full(W234), full(Wab), full(B14), full(Bab),
            full(lin_w), full(lb),
        ],
        out_specs=pl.BlockSpec((1, lin_w.shape[1]), lambda i: (0, 0)),
        out_shape=jax.ShapeDtypeStruct((1, lin_w.shape[1]), jnp.float32),
        scratch_shapes=[pltpu.VMEM((1, 13 * _BN), jnp.float32)],
    )(x, mask, W1, W234, Wab, B14, Bab, lin_w, lb)
    return out[0]
